# baseline (speedup 1.0000x reference)
import functools

import jax
import jax.numpy as jnp
from jax import lax
from jax.experimental import pallas as pl
from jax.experimental.pallas import tpu as pltpu
from jax.experimental.pallas import tpu_sc as plsc

EMB = 64
SCALE = 8.0
LANES = 16
ROW_UNROLL = 8


@functools.lru_cache(maxsize=None)
def _build(l: int, b: int, vocab: int):
    info = plsc.get_sparse_core_info()
    nc, ns = info.num_cores, info.num_subcores
    nw = nc * ns
    gl, gb = 8, 4
    assert gl * gb == nw and l % gl == 0 and b % gb == 0
    tl, tb = l // gl, b // gb

    mesh = plsc.VectorSubcoreMesh(core_axis_name="c", subcore_axis_name="s")

    @functools.partial(
        pl.kernel,
        mesh=mesh,
        compiler_params=pltpu.CompilerParams(use_tc_tiling_on_sc=False),
        out_type=jax.ShapeDtypeStruct((l, b, EMB), jnp.float32),
        scratch_types=[
            pltpu.VMEM((tl, tb), jnp.int32),
            pltpu.VMEM((tb, EMB), jnp.float32),
            pltpu.VMEM((tb, EMB), jnp.float32),
            pltpu.VMEM((tb, EMB), jnp.float32),
            pltpu.VMEM((tb, EMB), jnp.float32),
            pltpu.SemaphoreType.DMA,
            pltpu.SemaphoreType.DMA,
            pltpu.SemaphoreType.DMA,
            pltpu.SemaphoreType.DMA,
        ],
    )
    def gather_scale(
        table_hbm, idx_hbm, out_hbm,
        idx_v, in0, in1, ou0, ou1, g0, g1, s0, s1,
    ):
        ins = (in0, in1)
        ous = (ou0, ou1)
        gsems = (g0, g1)
        ssems = (s0, s1)
        wid = lax.axis_index("s") * nc + lax.axis_index("c")
        l0 = (wid // gb) * tl
        b0 = (wid % gb) * tb
        pltpu.sync_copy(idx_hbm.at[pl.ds(l0, tl), pl.ds(b0, tb)], idx_v)

        def gather_start(c, bf):
            pltpu.async_copy(table_hbm.at[idx_v.at[c]], ins[bf], gsems[bf])

        gather_start(0, 0)
        gather_start(1, 1)

        for c in range(tl):
            bf = c % 2
            pltpu.make_async_copy(
                table_hbm.at[idx_v.at[c]], ins[bf], gsems[bf]
            ).wait()
            if c >= 2:
                pltpu.make_async_copy(
                    ous[bf], out_hbm.at[l0, pl.ds(b0, tb)], ssems[bf]
                ).wait()

            def scale_body(i, _, bf=bf):
                for r in range(ROW_UNROLL):
                    row = i * ROW_UNROLL + r
                    for j in range(EMB // LANES):
                        sl = pl.ds(j * LANES, LANES)
                        ous[bf][row, sl] = ins[bf][row, sl] * SCALE
                return 0

            lax.fori_loop(0, tb // ROW_UNROLL, scale_body, 0)

            if c + 2 < tl:
                gather_start(c + 2, bf)
            pltpu.async_copy(
                ous[bf], out_hbm.at[l0 + c, pl.ds(b0, tb)], ssems[bf]
            )

        for bf in range(2):
            pltpu.make_async_copy(
                ous[bf], out_hbm.at[l0, pl.ds(b0, tb)], ssems[bf]
            ).wait()

    return gather_scale


def kernel(src, SenEmbedding_dict, embedding_weight):
    l, b = src.shape
    vocab, emb = embedding_weight.shape
    fn = _build(l, b, vocab)
    return fn(embedding_weight, src.astype(jnp.int32))
